# baseline (speedup 1.0000x reference)
import functools

import jax
import jax.numpy as jnp
from jax import lax
from jax.experimental import pallas as pl
from jax.experimental.pallas import tpu as pltpu
from jax.experimental.pallas import tpu_sc as plsc

_CHUNK = 128


@functools.lru_cache(maxsize=None)
def _make_gather(N, V, D, NC, NS):
    NW = NC * NS
    n_per_w = N // NW
    n_chunks = n_per_w // _CHUNK
    mesh = plsc.VectorSubcoreMesh(core_axis_name="c", subcore_axis_name="s")

    @functools.partial(
        pl.kernel,
        mesh=mesh,
        out_type=jax.ShapeDtypeStruct((N, D), jnp.float32),
        scratch_types=[
            pltpu.VMEM((n_chunks, _CHUNK), jnp.int32),
            pltpu.VMEM((_CHUNK, D), jnp.float32),
            pltpu.SemaphoreType.DMA,
        ],
        compiler_params=pltpu.CompilerParams(use_tc_tiling_on_sc=False),
    )
    def k(table_hbm, xr_hbm, out_hbm, idx_v, rows_v, sem):
        wid = lax.axis_index("s") * NC + lax.axis_index("c")
        base = wid * n_per_w
        pltpu.sync_copy(xr_hbm.at[wid], idx_v)

        def body(j, carry):
            pltpu.async_copy(table_hbm.at[idx_v.at[j]], rows_v, sem).wait()
            pltpu.sync_copy(rows_v, out_hbm.at[pl.ds(base + j * _CHUNK, _CHUNK)])
            return carry

        lax.fori_loop(0, n_chunks, body, 0)

    return k


def kernel(x, table):
    B, L = x.shape
    V, D = table.shape
    N = B * L
    info = plsc.get_sparse_core_info()
    NC, NS = info.num_cores, info.num_subcores
    NW = NC * NS
    n_per_w = N // NW
    xr = x.reshape(NW, n_per_w // _CHUNK, _CHUNK).astype(jnp.int32)
    out = _make_gather(N, V, D, NC, NS)(table, xr)
    return out.reshape(B, L, D)


# speedup vs baseline: 1.0618x; 1.0618x over previous
import functools

import jax
import jax.numpy as jnp
from jax import lax
from jax.experimental import pallas as pl
from jax.experimental.pallas import tpu as pltpu
from jax.experimental.pallas import tpu_sc as plsc

_CHUNK = 128
_NBUF = 8
_LAG = 4


@functools.lru_cache(maxsize=None)
def _make_gather(N, V, D, NC, NS):
    NW = NC * NS
    n_per_w = N // NW
    n_chunks = n_per_w // _CHUNK
    n_groups = n_chunks // _NBUF
    assert n_chunks % _NBUF == 0 and n_groups >= 2
    mesh = plsc.VectorSubcoreMesh(core_axis_name="c", subcore_axis_name="s")

    @functools.partial(
        pl.kernel,
        mesh=mesh,
        out_type=jax.ShapeDtypeStruct((N, D), jnp.float32),
        scratch_types=[
            pltpu.VMEM((n_chunks, _CHUNK), jnp.int32),
            [pltpu.VMEM((_CHUNK, D), jnp.float32) for _ in range(_NBUF)],
            [pltpu.SemaphoreType.DMA for _ in range(_NBUF)],
            [pltpu.SemaphoreType.DMA for _ in range(_NBUF)],
        ],
        compiler_params=pltpu.CompilerParams(use_tc_tiling_on_sc=False),
    )
    def k(table_hbm, xr_hbm, out_hbm, idx_v, bufs, gsem, wsem):
        wid = lax.axis_index("s") * NC + lax.axis_index("c")
        base = wid * n_per_w
        pltpu.sync_copy(xr_hbm.at[wid], idx_v)

        def gather(i, b):
            return pltpu.make_async_copy(
                table_hbm.at[idx_v.at[i]], bufs[b], gsem[b])

        def write(i, b):
            return pltpu.make_async_copy(
                bufs[b], out_hbm.at[pl.ds(base + i * _CHUNK, _CHUNK)], wsem[b])

        for b in range(_LAG):
            gather(b, b).start()

        def step(i, b, first_group):
            gather(i, b).wait()
            write(i, b).start()
            bg = (b + _LAG) % _NBUF
            if not (first_group and b < _LAG):
                write(i - _LAG, bg).wait()
            gather(i + _LAG, bg).start()

        for b in range(_NBUF):
            step(b, b, True)

        def group(g, carry):
            for b in range(_NBUF):
                step(g * _NBUF + b, b, False)
            return carry

        lax.fori_loop(1, n_groups - 1, group, 0)

        i0 = (n_groups - 1) * _NBUF
        for b in range(_NBUF):
            i = i0 + b
            gather(i, b).wait()
            write(i, b).start()
            if b < _NBUF - _LAG:
                write(i - _LAG, (b + _LAG) % _NBUF).wait()
                gather(i + _LAG, (b + _LAG) % _NBUF).start()
        for b in range(_NBUF):
            write(i0 + b, b).wait()

    return k


def kernel(x, table):
    B, L = x.shape
    V, D = table.shape
    N = B * L
    info = plsc.get_sparse_core_info()
    NC, NS = info.num_cores, info.num_subcores
    NW = NC * NS
    n_per_w = N // NW
    xr = x.reshape(NW, n_per_w // _CHUNK, _CHUNK).astype(jnp.int32)
    out = _make_gather(N, V, D, NC, NS)(table, xr)
    return out.reshape(B, L, D)
